# Initial kernel scaffold; baseline (speedup 1.0000x reference)
import jax
import jax.numpy as jnp
from jax.experimental import pallas as pl


def kernel(x, batch, W1, b1, W2, b2, W3, b3):
    raise NotImplementedError("write your pallas kernel here")


# speedup vs baseline: 2.6444x; 2.6444x over previous
import functools

import jax
import jax.numpy as jnp
from jax import lax
from jax.experimental import pallas as pl
from jax.experimental.pallas import tpu as pltpu
from jax.experimental.pallas import tpu_sc as plsc

N = 100000
D = 128
H = 256
O = 10
G = 256

NC = 2
NS = 16
NW = NC * NS
RPW = N // NW
CHUNK = 625
NCHUNK = RPW // CHUNK
LANES = 16


def _sc_segment_sum(x3, batch2):
  mesh = plsc.VectorSubcoreMesh(core_axis_name="c", subcore_axis_name="s")

  @functools.partial(
      pl.kernel,
      out_type=[
          jax.ShapeDtypeStruct((NW, G * D), jnp.float32),
          jax.ShapeDtypeStruct((NW, G * LANES), jnp.float32),
      ],
      mesh=mesh,
      compiler_params=pltpu.CompilerParams(use_tc_tiling_on_sc=False),
      scratch_types=[
          pltpu.VMEM((G * D,), jnp.float32),
          pltpu.VMEM((G * LANES,), jnp.float32),
          pltpu.VMEM((RPW + LANES,), jnp.int32),
          pltpu.VMEM((CHUNK, D), jnp.float32),
      ],
  )
  def seg_kernel(x_hbm, b_hbm, part_hbm, cnt_hbm, acc, cnt, bat, xbuf):
    wid = lax.axis_index("s") * NC + lax.axis_index("c")
    zeros = jnp.zeros((LANES,), jnp.float32)
    ones = jnp.ones((LANES,), jnp.float32)

    def zero_acc(i, _):
      acc[pl.ds(i * LANES, LANES)] = zeros
      return _
    lax.fori_loop(0, G * D // LANES, zero_acc, 0)

    def zero_cnt(i, _):
      cnt[pl.ds(i * LANES, LANES)] = zeros
      return _
    lax.fori_loop(0, G, zero_cnt, 0)

    pltpu.sync_copy(b_hbm.at[wid], bat.at[pl.ds(0, RPW)])

    def do_chunk(c, _):
      pltpu.sync_copy(x_hbm.at[wid, pl.ds(c * CHUNK, CHUNK)], xbuf)

      def do_row(r, _):
        seg = bat[pl.ds(c * CHUNK + r, LANES)][0]
        base = seg * D
        for j in range(D // LANES):
          v = xbuf[r, pl.ds(j * LANES, LANES)]
          plsc.addupdate(acc.at[pl.ds(base + j * LANES, LANES)], v)
        plsc.addupdate(cnt.at[pl.ds(seg * LANES, LANES)], ones)
        return _
      lax.fori_loop(0, CHUNK, do_row, 0)
      return _
    lax.fori_loop(0, NCHUNK, do_chunk, 0)

    pltpu.sync_copy(acc, part_hbm.at[wid])
    pltpu.sync_copy(cnt, cnt_hbm.at[wid])

  return seg_kernel(x3, batch2)


def _tc_pool_mlp(part, cnt, W1, b1, W2, b2, W3, b3):
  def body(part_ref, cnt_ref, w1_ref, b1_ref, w2_ref, b2_ref, w3_ref, b3_ref,
           out_ref):
    sums = jnp.sum(part_ref[...], axis=0)
    counts = jnp.sum(cnt_ref[...], axis=(0, 2)) * (1.0 / LANES)
    pooled = sums / jnp.clip(counts, 1.0)[:, None]
    h = jnp.maximum(
        jnp.dot(pooled, w1_ref[...], preferred_element_type=jnp.float32)
        + b1_ref[...], 0.0)
    h = jnp.maximum(
        jnp.dot(h, w2_ref[...], preferred_element_type=jnp.float32)
        + b2_ref[...], 0.0)
    out_ref[...] = (
        jnp.dot(h, w3_ref[...], preferred_element_type=jnp.float32)
        + b3_ref[...])

  return pl.pallas_call(
      body,
      out_shape=jax.ShapeDtypeStruct((G, O), jnp.float32),
  )(part, cnt, W1, b1.reshape(1, H), W2, b2.reshape(1, H), W3,
    b3.reshape(1, O))


def kernel(x, batch, W1, b1, W2, b2, W3, b3):
  x3 = x.reshape(NW, RPW, D)
  batch2 = batch.astype(jnp.int32).reshape(NW, RPW)
  part, cnt = _sc_segment_sum(x3, batch2)
  return _tc_pool_mlp(part.reshape(NW, G, D), cnt.reshape(NW, G, LANES),
                      W1, b1, W2, b2, W3, b3)


# speedup vs baseline: 3.1657x; 1.1971x over previous
import functools

import jax
import jax.numpy as jnp
from jax import lax
from jax.experimental import pallas as pl
from jax.experimental.pallas import tpu as pltpu
from jax.experimental.pallas import tpu_sc as plsc

N = 100000
D = 128
H = 256
O = 10
G = 256

NC = 2
NS = 16
NW = NC * NS
RPW = N // NW
LANES = 16
GA = G + 1

GROUPS = (RPW + LANES - 1) // LANES
CHUNK = 320
GPC = CHUNK // LANES
NFULL = (GROUPS * LANES) // CHUNK
TAIL_ROWS = GROUPS * LANES - NFULL * CHUNK
TAIL_VALID = RPW - NFULL * CHUNK
TAIL_G = TAIL_ROWS // LANES
BAT_PAD = 3152


def _sc_segment_sum(x3, batch2):
  mesh = plsc.VectorSubcoreMesh(core_axis_name="c", subcore_axis_name="s")

  @functools.partial(
      pl.kernel,
      out_type=[
          jax.ShapeDtypeStruct((NW, G * D), jnp.float32),
          jax.ShapeDtypeStruct((NW, G, LANES), jnp.float32),
      ],
      mesh=mesh,
      compiler_params=pltpu.CompilerParams(use_tc_tiling_on_sc=False,
                                           needs_layout_passes=False),
      scratch_types=[
          pltpu.VMEM((GA * D,), jnp.float32),
          pltpu.VMEM((GA, LANES), jnp.float32),
          pltpu.VMEM((BAT_PAD,), jnp.int32),
          pltpu.VMEM((CHUNK, D), jnp.float32),
      ],
  )
  def seg_kernel(x_hbm, b_hbm, part_hbm, cnt_hbm, acc, cnt, bat, xbuf):
    wid = lax.axis_index("s") * NC + lax.axis_index("c")
    zeros = jnp.zeros((LANES,), jnp.float32)
    ones = jnp.ones((LANES,), jnp.float32)
    lane_iota = lax.iota(jnp.int32, LANES)
    sentinel = jnp.full((LANES,), G, jnp.int32)

    def zero_acc(i, _):
      acc[pl.ds(i * LANES, LANES)] = zeros
      return _
    lax.fori_loop(0, GA * D // LANES, zero_acc, 0)

    def zero_cnt(i, _):
      cnt[i, :] = zeros
      return _
    lax.fori_loop(0, GA, zero_cnt, 0)

    pltpu.sync_copy(b_hbm.at[wid], bat.at[pl.ds(0, RPW)])
    bat[pl.ds(RPW, LANES)] = sentinel

    def do_group(g, _):
      segv = bat[pl.ds(g * LANES, LANES)]
      basev = segv * D
      plsc.addupdate_scatter(cnt, [segv, lane_iota], ones)
      r0 = (g * LANES) % CHUNK
      for j in range(LANES):
        base = basev[j]
        r = r0 + j
        for k in range(D // LANES):
          v = xbuf[r, pl.ds(k * LANES, LANES)]
          plsc.addupdate(acc.at[pl.ds(base + k * LANES, LANES)], v)
      return _

    def do_chunk(c, _):
      pltpu.sync_copy(x_hbm.at[wid, pl.ds(c * CHUNK, CHUNK)], xbuf)
      lax.fori_loop(c * GPC, (c + 1) * GPC, do_group, 0)
      return _
    lax.fori_loop(0, NFULL, do_chunk, 0)

    pltpu.sync_copy(x_hbm.at[wid, pl.ds(NFULL * CHUNK, TAIL_VALID)],
                    xbuf.at[pl.ds(0, TAIL_VALID)])
    lax.fori_loop(NFULL * GPC, NFULL * GPC + TAIL_G, do_group, 0)

    pltpu.sync_copy(acc.at[pl.ds(0, G * D)], part_hbm.at[wid])
    pltpu.sync_copy(cnt.at[pl.ds(0, G)], cnt_hbm.at[wid])

  return seg_kernel(x3, batch2)


def _tc_pool_mlp(part, cnt, W1, b1, W2, b2, W3, b3):
  def body(part_ref, cnt_ref, w1_ref, b1_ref, w2_ref, b2_ref, w3_ref, b3_ref,
           out_ref):
    sums = jnp.sum(part_ref[...], axis=0)
    counts = jnp.sum(cnt_ref[...], axis=(0, 2))
    pooled = sums / jnp.clip(counts, 1.0)[:, None]
    h = jnp.maximum(
        jnp.dot(pooled, w1_ref[...], preferred_element_type=jnp.float32)
        + b1_ref[...], 0.0)
    h = jnp.maximum(
        jnp.dot(h, w2_ref[...], preferred_element_type=jnp.float32)
        + b2_ref[...], 0.0)
    out_ref[...] = (
        jnp.dot(h, w3_ref[...], preferred_element_type=jnp.float32)
        + b3_ref[...])

  return pl.pallas_call(
      body,
      out_shape=jax.ShapeDtypeStruct((G, O), jnp.float32),
  )(part, cnt, W1, b1.reshape(1, H), W2, b2.reshape(1, H), W3,
    b3.reshape(1, O))


def kernel(x, batch, W1, b1, W2, b2, W3, b3):
  x3 = x.reshape(NW, RPW, D)
  batch2 = batch.astype(jnp.int32).reshape(NW, RPW)
  part, cnt = _sc_segment_sum(x3, batch2)
  return _tc_pool_mlp(part.reshape(NW, G, D), cnt, W1, b1, W2, b2, W3, b3)
